# baseline (device time: 96988 ns/iter reference)
import jax
import jax.numpy as jnp
from jax import lax
from jax.experimental import pallas as pl
from jax.experimental.pallas import tpu as pltpu

C = 288
E_LOC = 4
D = 1024
F = 2048
NFH = 4
FH = F // NFH
C4 = E_LOC * C
T = 2048


def _body(x_ref, idxr_ref, w1_ref, w2_ref, out_ref,
          send_buf, recv_buf, a_buf, y_acc, w1b, w2b,
          ssem, rsem, bssem, brsem):
    e = pl.program_id(0)
    fh = pl.program_id(1)
    my_x = lax.axis_index("x")
    my_y = lax.axis_index("y")
    peer = (1 - my_x, my_y)

    def disp_rdma(k):
        return pltpu.make_async_remote_copy(
            src_ref=send_buf.at[pl.ds(k * C, C)],
            dst_ref=recv_buf.at[pl.ds(k * C, C)],
            send_sem=ssem.at[k],
            recv_sem=rsem.at[k],
            device_id=peer,
            device_id_type=pl.DeviceIdType.MESH,
        )

    def back_rdma(k):
        return pltpu.make_async_remote_copy(
            src_ref=recv_buf.at[pl.ds(k * C, C)],
            dst_ref=send_buf.at[pl.ds(k * C, C)],
            send_sem=bssem.at[k],
            recv_sem=brsem.at[k],
            device_id=peer,
            device_id_type=pl.DeviceIdType.MESH,
        )

    @pl.when((e == 0) & (fh == 0))
    def _():
        barrier_sem = pltpu.get_barrier_semaphore()
        pl.semaphore_signal(
            barrier_sem, inc=1,
            device_id=peer, device_id_type=pl.DeviceIdType.MESH,
        )
        pl.semaphore_wait(barrier_sem, 1)

        xv = x_ref[...]
        idxr = idxr_ref[...]
        for k in range(E_LOC):
            iot = lax.broadcasted_iota(jnp.int32, (C, T), 0) + (C4 + k * C)
            m = (iot == jnp.broadcast_to(idxr, (C, T))).astype(jnp.bfloat16)
            send_buf[pl.ds(k * C, C), :] = jnp.dot(
                m, xv, preferred_element_type=jnp.float32
            ).astype(jnp.bfloat16)
            disp_rdma(k).start()
        for k in range(E_LOC):
            iot = lax.broadcasted_iota(jnp.int32, (C, T), 0) + k * C
            m = (iot == jnp.broadcast_to(idxr, (C, T))).astype(jnp.bfloat16)
            out_ref[pl.ds(k * C, C), :] = jnp.dot(
                m, xv, preferred_element_type=jnp.float32
            ).astype(jnp.bfloat16)

    for k in range(E_LOC):
        @pl.when((e == k) & (fh == 0))
        def _(k=k):
            disp_rdma(k).wait_recv()

    @pl.when(fh == 0)
    def _():
        a_buf[:C, :] = out_ref[pl.ds(e * C, C), :]
        a_buf[C:, :] = recv_buf[pl.ds(e * C, C), :]

    w1b[...] = w1_ref[0].astype(jnp.bfloat16)
    w2b[...] = w2_ref[0].astype(jnp.bfloat16)
    h = jnp.maximum(
        jnp.dot(a_buf[...], w1b[...], preferred_element_type=jnp.float32), 0
    ).astype(jnp.bfloat16)
    part = jnp.dot(h, w2b[...], preferred_element_type=jnp.float32)

    @pl.when(fh == 0)
    def _():
        y_acc[...] = part.astype(jnp.bfloat16)

    @pl.when((fh > 0) & (fh < NFH - 1))
    def _():
        y_acc[...] = (y_acc[...] + part).astype(jnp.bfloat16)

    @pl.when(fh == NFH - 1)
    def _():
        y = y_acc[...] + part
        recv_buf[pl.ds(e * C, C), :] = y[C:].astype(jnp.bfloat16)
        out_ref[pl.ds(e * C, C), :] = y[:C].astype(jnp.bfloat16)

    for k in range(E_LOC):
        @pl.when((e == k) & (fh == NFH - 1))
        def _(k=k):
            back_rdma(k).start()
            if k > 0:
                back_rdma(k - 1).wait_recv()
                out_ref[pl.ds(C4 + (k - 1) * C, C), :] = \
                    send_buf[pl.ds((k - 1) * C, C), :]

    @pl.when((e == E_LOC - 1) & (fh == NFH - 1))
    def _():
        back_rdma(E_LOC - 1).wait_recv()
        out_ref[pl.ds(C4 + (E_LOC - 1) * C, C), :] = \
            send_buf[pl.ds((E_LOC - 1) * C, C), :]
        for k in range(E_LOC):
            disp_rdma(k).wait_send()
            back_rdma(k).wait_send()


def kernel(x, assign, W1, W2):
    px = lax.axis_index("x")
    assign = assign.astype(jnp.int32)
    oh = assign[:, None] == jnp.arange(8, dtype=jnp.int32)[None, :]
    pos = jnp.cumsum(oh.astype(jnp.int32), axis=0) - 1
    pos_t = jnp.sum(jnp.where(oh, pos, 0), axis=1)
    valid = pos_t < C
    half = (assign // 4 != px).astype(jnp.int32)
    idx = jnp.where(valid, half * C4 + (assign % 4) * C + pos_t, 2 * C4)

    full = lambda e, fh: (0, 0)
    y_glob = pl.pallas_call(
        _body,
        grid=(E_LOC, NFH),
        in_specs=[
            pl.BlockSpec((T, D), full),
            pl.BlockSpec((1, T), full),
            pl.BlockSpec((1, D, FH), lambda e, fh: (e, 0, fh)),
            pl.BlockSpec((1, FH, D), lambda e, fh: (e, fh, 0)),
        ],
        out_specs=pl.BlockSpec((2 * C4, D), full),
        out_shape=jax.ShapeDtypeStruct((2 * C4, D), jnp.bfloat16),
        scratch_shapes=[
            pltpu.VMEM((C4, D), jnp.bfloat16),
            pltpu.VMEM((C4, D), jnp.bfloat16),
            pltpu.VMEM((2 * C, D), jnp.bfloat16),
            pltpu.VMEM((2 * C, D), jnp.bfloat16),
            pltpu.VMEM((D, FH), jnp.bfloat16),
            pltpu.VMEM((FH, D), jnp.bfloat16),
            pltpu.SemaphoreType.DMA((E_LOC,)),
            pltpu.SemaphoreType.DMA((E_LOC,)),
            pltpu.SemaphoreType.DMA((E_LOC,)),
            pltpu.SemaphoreType.DMA((E_LOC,)),
        ],
        compiler_params=pltpu.CompilerParams(collective_id=0),
    )(x.astype(jnp.bfloat16), idx.reshape(1, T), W1, W2)

    tok = jnp.full((2 * C4,), T, jnp.int32).at[idx].set(
        jnp.arange(T, dtype=jnp.int32), mode="drop"
    )
    out = jnp.zeros((T, D), jnp.bfloat16).at[tok].set(y_glob, mode="drop")
    return out


# device time: 75740 ns/iter; 1.2805x vs baseline; 1.2805x over previous
import jax
import jax.numpy as jnp
from jax import lax
from jax.experimental import pallas as pl
from jax.experimental.pallas import tpu as pltpu

C = 288
E_LOC = 4
D = 1024
F = 2048
NFH = 4
FH = F // NFH
C4 = E_LOC * C
T = 2048
NG = E_LOC * NFH + 1


def _body(x_ref, idxr_ref, idxc_ref, w1_ref, w2_ref, out_ref,
          local_buf, send_buf, recv_buf,
          a_buf, y_acc, w1b, w2b, ssem, rsem, bssem, brsem):
    g = pl.program_id(0)
    s = jnp.maximum(g - 1, 0)
    e = s // NFH
    fh = s % NFH
    my_x = lax.axis_index("x")
    my_y = lax.axis_index("y")
    peer = (1 - my_x, my_y)
    my_off = my_x * C4
    peer_off = (1 - my_x) * C4

    def disp_rdma(k):
        return pltpu.make_async_remote_copy(
            src_ref=send_buf.at[pl.ds(k * C, C)],
            dst_ref=recv_buf.at[pl.ds(k * C, C)],
            send_sem=ssem.at[k],
            recv_sem=rsem.at[k],
            device_id=peer,
            device_id_type=pl.DeviceIdType.MESH,
        )

    def back_rdma(k):
        return pltpu.make_async_remote_copy(
            src_ref=recv_buf.at[pl.ds(k * C, C)],
            dst_ref=send_buf.at[pl.ds(k * C, C)],
            send_sem=bssem.at[k],
            recv_sem=brsem.at[k],
            device_id=peer,
            device_id_type=pl.DeviceIdType.MESH,
        )

    @pl.when(g == 0)
    def _():
        barrier_sem = pltpu.get_barrier_semaphore()
        pl.semaphore_signal(
            barrier_sem, inc=1,
            device_id=peer, device_id_type=pl.DeviceIdType.MESH,
        )
        pl.semaphore_wait(barrier_sem, 1)

        xv = x_ref[...]
        idxr = idxr_ref[...]
        for k in range(E_LOC):
            iot = lax.broadcasted_iota(jnp.int32, (C, T), 0) + (peer_off + k * C)
            m = (iot == jnp.broadcast_to(idxr, (C, T))).astype(jnp.bfloat16)
            send_buf[pl.ds(k * C, C), :] = jnp.dot(
                m, xv, preferred_element_type=jnp.float32
            ).astype(jnp.bfloat16)
            disp_rdma(k).start()
        for k in range(E_LOC):
            iot = lax.broadcasted_iota(jnp.int32, (C, T), 0) + (my_off + k * C)
            m = (iot == jnp.broadcast_to(idxr, (C, T))).astype(jnp.bfloat16)
            local_buf[pl.ds(k * C, C), :] = jnp.dot(
                m, xv, preferred_element_type=jnp.float32
            ).astype(jnp.bfloat16)

    @pl.when(g < NG - 1)
    def _():
        slot = g % 2
        w1b[slot, :, :] = w1_ref[0].astype(jnp.bfloat16)
        w2b[slot, :, :] = w2_ref[0].astype(jnp.bfloat16)

    for k in range(E_LOC):
        @pl.when(g == NFH * k + 1)
        def _(k=k):
            disp_rdma(k).wait_recv()
            a_buf[:C, :] = local_buf[pl.ds(k * C, C), :]
            a_buf[C:, :] = recv_buf[pl.ds(k * C, C), :]

    @pl.when(g >= 1)
    def _():
        slot = (g + 1) % 2
        h = jnp.maximum(
            jnp.dot(a_buf[...], w1b[slot], preferred_element_type=jnp.float32),
            0.0,
        ).astype(jnp.bfloat16)
        part = jnp.dot(h, w2b[slot], preferred_element_type=jnp.float32)

        @pl.when(fh == 0)
        def _():
            y_acc[...] = part.astype(jnp.bfloat16)

        @pl.when((fh > 0) & (fh < NFH - 1))
        def _():
            y_acc[...] = (y_acc[...] + part).astype(jnp.bfloat16)

        @pl.when(fh == NFH - 1)
        def _():
            y = y_acc[...] + part
            recv_buf[pl.ds(e * C, C), :] = y[C:].astype(jnp.bfloat16)
            local_buf[pl.ds(e * C, C), :] = y[:C].astype(jnp.bfloat16)

    for k in range(E_LOC):
        @pl.when(g == NFH * (k + 1))
        def _(k=k):
            back_rdma(k).start()

    @pl.when(g == NG - 1)
    def _():
        idxc = idxc_ref[...]
        iot = lax.broadcasted_iota(jnp.int32, (T, C4), 1) + my_off
        gm = (iot == jnp.broadcast_to(idxc, (T, C4))).astype(jnp.bfloat16)
        out_ref[...] = jnp.dot(
            gm, local_buf[...], preferred_element_type=jnp.float32
        ).astype(jnp.bfloat16)
        for k in range(E_LOC):
            back_rdma(k).wait_recv()
        iot = lax.broadcasted_iota(jnp.int32, (T, C4), 1) + peer_off
        gb = (iot == jnp.broadcast_to(idxc, (T, C4))).astype(jnp.bfloat16)
        out_ref[...] += jnp.dot(
            gb, send_buf[...], preferred_element_type=jnp.float32
        ).astype(jnp.bfloat16)
        for k in range(E_LOC):
            disp_rdma(k).wait_send()
            back_rdma(k).wait_send()


def kernel(x, assign, W1, W2):
    oh = assign[:, None] == jnp.arange(8, dtype=assign.dtype)[None, :]
    pos = jnp.cumsum(oh.astype(jnp.int32), axis=0) - 1
    pos_t = jnp.sum(jnp.where(oh, pos, 0), axis=1)
    valid = pos_t < C
    idx = jnp.where(valid, assign.astype(jnp.int32) * C + pos_t, 8 * C)

    def wstep(g):
        s = jnp.minimum(g, NG - 2)
        return s // NFH, s % NFH

    out = pl.pallas_call(
        _body,
        grid=(NG,),
        in_specs=[
            pl.BlockSpec((T, D), lambda g: (0, 0)),
            pl.BlockSpec((1, T), lambda g: (0, 0)),
            pl.BlockSpec((T, 1), lambda g: (0, 0)),
            pl.BlockSpec(
                (1, D, FH), lambda g: (wstep(g)[0], 0, wstep(g)[1])
            ),
            pl.BlockSpec(
                (1, FH, D), lambda g: (wstep(g)[0], wstep(g)[1], 0)
            ),
        ],
        out_specs=pl.BlockSpec((T, D), lambda g: (0, 0)),
        out_shape=jax.ShapeDtypeStruct((T, D), jnp.bfloat16),
        scratch_shapes=[
            pltpu.VMEM((C4, D), jnp.bfloat16),
            pltpu.VMEM((C4, D), jnp.bfloat16),
            pltpu.VMEM((C4, D), jnp.bfloat16),
            pltpu.VMEM((2 * C, D), jnp.bfloat16),
            pltpu.VMEM((2 * C, D), jnp.bfloat16),
            pltpu.VMEM((2, D, FH), jnp.bfloat16),
            pltpu.VMEM((2, FH, D), jnp.bfloat16),
            pltpu.SemaphoreType.DMA((E_LOC,)),
            pltpu.SemaphoreType.DMA((E_LOC,)),
            pltpu.SemaphoreType.DMA((E_LOC,)),
            pltpu.SemaphoreType.DMA((E_LOC,)),
        ],
        compiler_params=pltpu.CompilerParams(collective_id=0),
    )(
        x.astype(jnp.bfloat16),
        idx.reshape(1, T),
        idx.reshape(T, 1),
        W1,
        W2,
    )
    return out
